# baseline (speedup 1.0000x reference)
import functools
import math

import jax
import jax.numpy as jnp
from jax import lax
from jax.experimental import pallas as pl
from jax.experimental.pallas import tpu as pltpu
from jax.experimental.pallas import tpu_sc as plsc

D = 64
SCALE = math.sqrt(D)
NC = 2
NS = 16
NW = NC * NS
L = 16
VOCAB = 1000000
NPAIR = VOCAB // 2
NFULL = VOCAB // 128
PITCH = 136

_mesh = lambda: plsc.VectorSubcoreMesh(core_axis_name="c", subcore_axis_name="s")
_params = lambda: pltpu.CompilerParams(
    use_tc_tiling_on_sc=True, needs_layout_passes=False)


def _transpose_kernel():
    @functools.partial(
        pl.kernel,
        mesh=_mesh(),
        compiler_params=_params(),
        out_type=jax.ShapeDtypeStruct((NPAIR, 128), jnp.float32),
        scratch_types=(
            [pltpu.VMEM((D, PITCH), jnp.float32)] * 3
            + [pltpu.VMEM((D, 128), jnp.float32)] * 3
            + [pltpu.SemaphoreType.DMA] * 6
        ),
    )
    def tk(tabT, tail, tabP, *rest):
        ins = rest[0:3]
        outs = rest[3:6]
        gis = rest[6:9]
        gos = rest[9:12]
        in0, out0 = ins[0], outs[0]
        wid = lax.axis_index("s") * NC + lax.axis_index("c")

        n_my = (NFULL - 1 - wid) // NW + 1

        def start_in(t, slot):
            b = t * NW + wid
            pltpu.async_copy(
                tabT.at[:, pl.ds(b * 128, 128)],
                ins[slot].at[:, pl.ds(0, 128)],
                gis[slot],
            )

        def wait_in(slot):
            pltpu.make_async_copy(
                tabT.at[:, pl.ds(0, 128)],
                ins[slot].at[:, pl.ds(0, 128)],
                gis[slot],
            ).wait()

        def start_out(t, slot):
            b = t * NW + wid
            pltpu.async_copy(outs[slot], tabP.at[pl.ds(b * 64, 64)], gos[slot])

        def wait_out(slot):
            pltpu.make_async_copy(
                outs[slot], tabP.at[pl.ds(0, 64)], gos[slot]).wait()

        def compute(slot):
            src = ins[slot]
            dst = outs[slot]
            rows8 = [lax.iota(jnp.int32, L) + (cb % 4) * L for cb in range(4)]
            zero = jnp.full((L,), 0, jnp.int32)

            @plsc.parallel_loop(0, D, unroll=8)
            def _(p):
                cols0 = zero + 2 * p
                cols1 = cols0 + 1
                for cb in range(8):
                    v = plsc.load_gather(
                        src, [rows8[cb % 4], cols0 if cb < 4 else cols1])
                    dst[p, pl.ds(cb * L, L)] = v * SCALE

        for k in range(3):
            @pl.when(n_my > k)
            def _(k=k):
                start_in(k, k)

        def body(t, carry):
            slot = lax.rem(t, 3)

            def stage(s):
                @pl.when(t >= 3)
                def _():
                    wait_out(s)
                wait_in(s)
                compute(s)
                start_out(t, s)

                @pl.when(t + 3 < n_my)
                def _():
                    start_in(t + 3, s)

            for s in range(3):
                @pl.when(slot == s)
                def _():
                    stage(s)
            return carry

        lax.fori_loop(0, n_my, body, 0)
        for k in range(3):
            @pl.when(n_my > k)
            def _(k=k):
                wait_out(k)

        @pl.when(wid == (NFULL % NW))
        def _():
            pltpu.sync_copy(tail, out0.at[pl.ds(0, 32)])
            pltpu.sync_copy(
                out0.at[pl.ds(0, 32)], tabP.at[pl.ds(NFULL * 64, 32)])

    return tk


def _gather_kernel():
    @functools.partial(
        pl.kernel,
        mesh=_mesh(),
        compiler_params=_params(),
        out_type=jax.ShapeDtypeStruct((200, D, 4096), jnp.float32),
        scratch_types=(
            [pltpu.VMEM((8, 128), jnp.int32)] * 2
            + [pltpu.VMEM((128, PITCH), jnp.float32)] * 3
            + [pltpu.VMEM((D, 128), jnp.float32)] * 2
            + [pltpu.SemaphoreType.DMA] * 6
        ),
    )
    def gk(xT, tabP, out3, *rest):
        idxv, idx2 = rest[0:2]
        gs = rest[2:5]
        bs = rest[5:7]
        sgs = rest[7:10]
        sbs = rest[10:12]
        wid = lax.axis_index("s") * NC + lax.axis_index("c")
        col0 = wid * 128

        def load_idx_block(a):
            pltpu.sync_copy(xT.at[pl.ds(a * 8, 8), pl.ds(col0, 128)], idxv)

            def halve(r, carry):
                for q in range(8):
                    sl = pl.ds(q * L, L)
                    idx2[r, sl] = lax.shift_right_logical(idxv[r, sl], 1)
                return carry

            lax.fori_loop(0, 8, halve, 0)

        def start_gather(r, slot):
            pltpu.async_copy(
                tabP.at[idx2.at[r]], gs[slot].at[:, pl.ds(0, 128)], sgs[slot])

        def wait_gather(slot):
            pltpu.make_async_copy(
                tabP.at[idx2.at[0]], gs[slot].at[:, pl.ds(0, 128)],
                sgs[slot]).wait()

        def assemble(r, gslot, oslot):
            g = gs[gslot]
            b = bs[oslot]
            for jb in range(8):
                par = (idxv[r, pl.ds(jb * L, L)] & 1) * D
                rows = lax.iota(jnp.int32, L) + jb * L

                @plsc.parallel_loop(0, D, unroll=8)
                def _(c):
                    v = plsc.load_gather(g, [rows, par + c])
                    b[c, pl.ds(jb * L, L)] = v

        def start_out(s1, slot):
            pltpu.async_copy(
                bs[slot], out3.at[s1].at[:, pl.ds(col0, 128)], sbs[slot])

        def wait_out(slot):
            pltpu.make_async_copy(
                bs[slot], out3.at[0].at[:, pl.ds(col0, 128)], sbs[slot]).wait()

        def block(a, carry):
            load_idx_block(a)
            for k in range(3):
                start_gather(k, k)

            def srow(r, carry2):
                gslot = lax.rem(r, 3)
                oslot = lax.rem(r, 2)

                def stage(gsl, osl):
                    wait_gather(gsl)

                    @pl.when(r >= 2)
                    def _():
                        wait_out(osl)
                    assemble(r, gsl, osl)
                    start_out(a * 8 + r, osl)

                    @pl.when(r + 3 < 8)
                    def _():
                        start_gather(r + 3, gsl)

                for gsl in range(3):
                    @pl.when(gslot == gsl)
                    def _(gsl=gsl):
                        for osl in range(2):
                            @pl.when(oslot == osl)
                            def _(osl=osl):
                                stage(gsl, osl)
                return carry2

            lax.fori_loop(0, 8, srow, 0)
            wait_out(0)
            wait_out(1)
            return carry

        lax.fori_loop(0, 25, block, 0)

    return gk


def kernel(x, table):
    xT = x.T.astype(jnp.int32)
    tabT = table.T
    tail = (table[VOCAB - 64:] * SCALE).reshape(32, 128)
    tabP = _transpose_kernel()(tabT, tail)
    out3 = _gather_kernel()(xT, tabP)
    return out3.transpose(2, 0, 1)


# speedup vs baseline: 1.5697x; 1.5697x over previous
import functools
import math

import jax
import jax.numpy as jnp
from jax import lax
from jax.experimental import pallas as pl
from jax.experimental.pallas import tpu as pltpu
from jax.experimental.pallas import tpu_sc as plsc

D = 64
SCALE = math.sqrt(D)
NC = 2
NS = 16
NW = NC * NS
VOCAB = 1000000
CH = 128
NBUF = 4

_mesh = lambda: plsc.VectorSubcoreMesh(core_axis_name="c", subcore_axis_name="s")
_params = lambda: pltpu.CompilerParams(
    use_tc_tiling_on_sc=True, needs_layout_passes=False)


def _gather_kernel(B):
    per_w = B // NW
    n_ch = per_w // CH

    @functools.partial(
        pl.kernel,
        mesh=_mesh(),
        compiler_params=_params(),
        out_type=jax.ShapeDtypeStruct((B, 2 * D), jnp.float32),
        scratch_types=(
            [pltpu.VMEM((n_ch, CH), jnp.int32)]
            + [pltpu.VMEM((CH, 2 * D), jnp.float32)] * NBUF
            + [pltpu.SemaphoreType.DMA] * (2 * NBUF)
        ),
    )
    def gk(xF, tabD, out2, *rest):
        idxv = rest[0]
        gs = rest[1:1 + NBUF]
        sg = rest[1 + NBUF:1 + 2 * NBUF]
        so = rest[1 + 2 * NBUF:1 + 3 * NBUF]
        wid = lax.axis_index("s") * NC + lax.axis_index("c")
        base = wid * per_w

        pltpu.sync_copy(xF.at[pl.ds(wid * n_ch, n_ch)], idxv)

        def start_gather(c, s):
            pltpu.async_copy(tabD.at[idxv.at[c]], gs[s], sg[s])

        def wait_gather(s):
            pltpu.make_async_copy(tabD.at[idxv.at[0]], gs[s], sg[s]).wait()

        def start_out(c, s):
            pltpu.async_copy(
                gs[s], out2.at[pl.ds(base + c * CH, CH)], so[s])

        def wait_out(s):
            pltpu.make_async_copy(
                gs[s], out2.at[pl.ds(base, CH)], so[s]).wait()

        for k in range(NBUF):
            start_gather(k, k)

        def body(t, carry):
            slot = lax.rem(t, NBUF)

            def stage(s):
                wait_gather(s)

                @pl.when(t >= NBUF)
                def _():
                    wait_out(s)
                start_out(t, s)

                @pl.when(t + NBUF < n_ch)
                def _():
                    start_gather(t + NBUF, s)

            for s in range(NBUF):
                @pl.when(slot == s)
                def _():
                    stage(s)
            return carry

        lax.fori_loop(0, n_ch, body, 0)
        for k in range(NBUF):
            wait_out(k)

    return gk


def kernel(x, table):
    S0, S1 = x.shape
    B = S0 * S1
    xF = x.reshape(B // CH, CH).astype(jnp.int32)
    tabD = jnp.pad(table, ((0, 0), (0, D))) * SCALE
    out2 = _gather_kernel(B)(xF, tabD)
    return out2[:, :D].reshape(S0, S1, D)
